# baseline (speedup 1.0000x reference)
import functools

import jax
import jax.numpy as jnp
import numpy as np
from jax import lax
from jax.experimental import pallas as pl
from jax.experimental.pallas import tpu as pltpu
from jax.experimental.pallas import tpu_sc as plsc

L = 16
NC, NS = 2, 16
NW = NC * NS
IDXROW = 128


def _bcast_lane(v, k):
    return lax.gather(
        v, jnp.full((L, 1), k, jnp.int32),
        lax.GatherDimensionNumbers(
            offset_dims=(), collapsed_slice_dims=(0,), start_index_map=(0,)),
        slice_sizes=(1,),
        mode=lax.GatherScatterMode.PROMISE_IN_BOUNDS)


def _lo_f32(w):
    return lax.bitcast_convert_type(w << 16, jnp.float32)


def _hi_f32(w):
    return lax.bitcast_convert_type(w & jnp.int32(-65536), jnp.float32)


@functools.lru_cache(maxsize=None)
def _build(batch, fields, vocab, dim):
    bw = batch // NW
    kstream = bw // IDXROW
    n_chunks = fields
    words = dim // 2
    max_idx = vocab - 1

    mesh = plsc.VectorSubcoreMesh(core_axis_name="c", subcore_axis_name="s")

    @functools.partial(
        pl.kernel,
        out_type=jax.ShapeDtypeStruct((batch, fields, dim), jnp.float32),
        mesh=mesh,
        compiler_params=pltpu.CompilerParams(use_tc_tiling_on_sc=False),
        scratch_types=[
            pltpu.VMEM((bw,), jnp.float32),
            pltpu.VMEM((bw,), jnp.float32),
            pltpu.VMEM((kstream, IDXROW), jnp.int32),
            pltpu.VMEM((kstream, IDXROW), jnp.int32),
            pltpu.VMEM((kstream, IDXROW), jnp.int32),
            pltpu.VMEM((kstream, IDXROW), jnp.int32),
            pltpu.VMEM((bw, words), jnp.int32),
            pltpu.VMEM((bw, words), jnp.int32),
            pltpu.VMEM((bw, words), jnp.int32),
            pltpu.VMEM((bw, words), jnp.int32),
            pltpu.VMEM((bw, 1, dim), jnp.float32),
            pltpu.VMEM((bw, 1, dim), jnp.float32),
            pltpu.SemaphoreType.DMA,
            pltpu.SemaphoreType.DMA,
        ],
    )
    def impl(cont_hbm, w_hbm, out_hbm, c0, c1, il0, ih0, il1, ih1,
             lo0, hi0, lo1, hi1, o0, o1, s0, s1):
        wid = lax.axis_index("s") * NC + lax.axis_index("c")
        b0 = wid * bw

        def prep(f, cv, ilv, ihv, lov, hiv, sem):
            @pl.when(f < n_chunks)
            def _():
                pltpu.sync_copy(cont_hbm.at[f, pl.ds(b0, bw)], cv)

                def idx_body(t, _):
                    x = cv[pl.ds(t * L, L)]
                    il = x.astype(jnp.int32)
                    w = x - il.astype(jnp.float32)
                    ih = jnp.minimum(il + 1, max_idx)
                    r = t // (IDXROW // L)
                    c = (t % (IDXROW // L)) * L
                    ilv[r, pl.ds(c, L)] = il << 2
                    ihv[r, pl.ds(c, L)] = ih << 2
                    cv[pl.ds(t * L, L)] = w
                    return 0

                lax.fori_loop(0, bw // L, idx_body, 0)
                for j in range(kstream):
                    pltpu.async_copy(
                        w_hbm.at[ilv.at[j]],
                        lov.at[pl.ds(j * IDXROW, IDXROW)], sem)
                    pltpu.async_copy(
                        w_hbm.at[ihv.at[j]],
                        hiv.at[pl.ds(j * IDXROW, IDXROW)], sem)

        def waitg(ilv, ihv, lov, hiv, sem):
            for j in range(kstream):
                pltpu.make_async_copy(
                    w_hbm.at[ilv.at[j]],
                    lov.at[pl.ds(j * IDXROW, IDXROW)], sem).wait()
                pltpu.make_async_copy(
                    w_hbm.at[ihv.at[j]],
                    hiv.at[pl.ds(j * IDXROW, IDXROW)], sem).wait()

        def blendout(f, cv, lov, hiv, ov):
            def blend_body(t, _):
                w16 = cv[pl.ds(t * L, L)]
                for k in range(L):
                    i = t * L + k
                    wv = _bcast_lane(w16, k)
                    rl = lov[i, pl.ds(0, L)]
                    rh = hiv[i, pl.ds(0, L)]
                    a0, a1 = _lo_f32(rl), _hi_f32(rl)
                    b0_, b1 = _lo_f32(rh), _hi_f32(rh)
                    ov[i, 0, pl.ds(0, L)] = a0 + wv * (b0_ - a0)
                    ov[i, 0, pl.ds(L, L)] = a1 + wv * (b1 - a1)
                return 0

            lax.fori_loop(0, bw // L, blend_body, 0)
            pltpu.sync_copy(ov, out_hbm.at[pl.ds(b0, bw), pl.ds(f, 1)])

        set0 = (c0, il0, ih0, lo0, hi0, s0)
        set1 = (c1, il1, ih1, lo1, hi1, s1)

        prep(0, *set0)

        def step(s, _):
            prep(2 * s + 1, *set1)
            waitg(*set0[1:])
            blendout(2 * s, c0, lo0, hi0, o0)
            prep(2 * s + 2, *set0)
            waitg(*set1[1:])
            blendout(2 * s + 1, c1, lo1, hi1, o1)
            return 0

        lax.fori_loop(0, n_chunks // 2, step, 0)

    return impl


def kernel(continuous_idx, W):
    batch, fields = continuous_idx.shape
    vocab, dim = W.shape
    impl = _build(batch, fields, vocab, dim)
    w_b = jnp.stack(
        [W[:, :dim // 2], W[:, dim // 2:]], axis=2,
    ).reshape(vocab, dim).astype(jnp.bfloat16)
    w_p = jnp.pad(w_b, ((0, 0), (0, 3 * dim)))
    w_i = lax.bitcast_convert_type(
        w_p.reshape(4 * vocab, dim // 2, 2), jnp.int32)
    return impl(continuous_idx.T, w_i)


# speedup vs baseline: 128.8869x; 128.8869x over previous
import functools

import jax
import jax.numpy as jnp
from jax import lax
from jax.experimental import pallas as pl
from jax.experimental.pallas import tpu as pltpu
from jax.experimental.pallas import tpu_sc as plsc

L = 16
NC, NS = 2, 16
NW = NC * NS
IDXROW = 128


def _bcast_lane(v, k):
    return lax.gather(
        v, jnp.full((L, 1), k, jnp.int32),
        lax.GatherDimensionNumbers(
            offset_dims=(), collapsed_slice_dims=(0,), start_index_map=(0,)),
        slice_sizes=(1,),
        mode=lax.GatherScatterMode.PROMISE_IN_BOUNDS)


@functools.lru_cache(maxsize=None)
def _build(batch, fields, vocab, dim):
    bw = batch // NW
    kstream = bw // IDXROW
    n_chunks = fields
    max_idx = vocab - 1

    mesh = plsc.VectorSubcoreMesh(core_axis_name="c", subcore_axis_name="s")

    @functools.partial(
        pl.kernel,
        out_type=jax.ShapeDtypeStruct((batch, fields, dim), jnp.float32),
        mesh=mesh,
        compiler_params=pltpu.CompilerParams(use_tc_tiling_on_sc=False),
        scratch_types=[
            pltpu.VMEM((bw,), jnp.float32),
            pltpu.VMEM((bw,), jnp.float32),
            pltpu.VMEM((kstream, IDXROW), jnp.int32),
            pltpu.VMEM((kstream, IDXROW), jnp.int32),
            pltpu.VMEM((kstream, IDXROW), jnp.int32),
            pltpu.VMEM((kstream, IDXROW), jnp.int32),
            pltpu.VMEM((bw, dim), jnp.float32),
            pltpu.VMEM((bw, dim), jnp.float32),
            pltpu.VMEM((bw, dim), jnp.float32),
            pltpu.VMEM((bw, dim), jnp.float32),
            pltpu.VMEM((bw, 1, dim), jnp.float32),
            pltpu.VMEM((bw, 1, dim), jnp.float32),
            pltpu.SemaphoreType.DMA,
            pltpu.SemaphoreType.DMA,
        ],
    )
    def impl(cont_hbm, w_hbm, out_hbm, c0, c1, il0, ih0, il1, ih1,
             lo0, hi0, lo1, hi1, o0, o1, s0, s1):
        wid = lax.axis_index("s") * NC + lax.axis_index("c")
        b0 = wid * bw

        def prep(f, cv, ilv, ihv, lov, hiv, sem):
            @pl.when(f < n_chunks)
            def _():
                pltpu.sync_copy(cont_hbm.at[f, pl.ds(b0, bw)], cv)

                def idx_body(t, _):
                    x = cv[pl.ds(t * L, L)]
                    il = x.astype(jnp.int32)
                    w = x - il.astype(jnp.float32)
                    ih = jnp.minimum(il + 1, max_idx)
                    r = t // (IDXROW // L)
                    c = (t % (IDXROW // L)) * L
                    ilv[r, pl.ds(c, L)] = il << 2
                    ihv[r, pl.ds(c, L)] = ih << 2
                    cv[pl.ds(t * L, L)] = w
                    return 0

                lax.fori_loop(0, bw // L, idx_body, 0)
                for j in range(kstream):
                    pltpu.async_copy(
                        w_hbm.at[ilv.at[j]],
                        lov.at[pl.ds(j * IDXROW, IDXROW)], sem)
                    pltpu.async_copy(
                        w_hbm.at[ihv.at[j]],
                        hiv.at[pl.ds(j * IDXROW, IDXROW)], sem)

        def waitg(ilv, ihv, lov, hiv, sem):
            for j in range(kstream):
                pltpu.make_async_copy(
                    w_hbm.at[ilv.at[j]],
                    lov.at[pl.ds(j * IDXROW, IDXROW)], sem).wait()
                pltpu.make_async_copy(
                    w_hbm.at[ihv.at[j]],
                    hiv.at[pl.ds(j * IDXROW, IDXROW)], sem).wait()

        def blendout(f, cv, lov, hiv, ov):
            def blend_body(t, _):
                w16 = cv[pl.ds(t * L, L)]
                for k in range(L):
                    i = t * L + k
                    wv = _bcast_lane(w16, k)
                    for d in range(dim // L):
                        lo = lov[i, pl.ds(d * L, L)]
                        hi = hiv[i, pl.ds(d * L, L)]
                        ov[i, 0, pl.ds(d * L, L)] = lo + wv * (hi - lo)
                return 0

            lax.fori_loop(0, bw // L, blend_body, 0)
            pltpu.sync_copy(ov, out_hbm.at[pl.ds(b0, bw), pl.ds(f, 1)])

        set0 = (c0, il0, ih0, lo0, hi0, s0)
        set1 = (c1, il1, ih1, lo1, hi1, s1)

        prep(0, *set0)

        def step(s, _):
            prep(2 * s + 1, *set1)
            waitg(*set0[1:])
            blendout(2 * s, c0, lo0, hi0, o0)
            prep(2 * s + 2, *set0)
            waitg(*set1[1:])
            blendout(2 * s + 1, c1, lo1, hi1, o1)
            return 0

        lax.fori_loop(0, n_chunks // 2, step, 0)

    return impl


def kernel(continuous_idx, W):
    batch, fields = continuous_idx.shape
    vocab, dim = W.shape
    impl = _build(batch, fields, vocab, dim)
    w_pad = jnp.pad(W, ((0, 0), (0, 3 * dim)))
    return impl(continuous_idx.T, w_pad.reshape(4 * vocab, dim))
